# baseline (speedup 1.0000x reference)
import math
import functools

import jax
import jax.numpy as jnp
from jax.experimental import pallas as pl
from jax.experimental.pallas import tpu as pltpu

_NT = 1000
_ROWS = 8
_COLS = 4 * 64 * 64


def _schedule_tables():
    steps = _NT + 1
    xs = jnp.linspace(0.0, float(_NT), steps, dtype=jnp.float32)
    acp = jnp.cos((xs / _NT + 0.008) / (1 + 0.008) * math.pi * 0.5) ** 2
    acp = acp / acp[0]
    betas = jnp.clip(1.0 - acp[1:] / acp[:-1], 0.0001, 0.9999)
    alphas_cumprod = jnp.cumprod(1.0 - betas, axis=0)
    c1 = jnp.sqrt(alphas_cumprod)
    c2 = jnp.sqrt(1.0 - alphas_cumprod)
    return c1, c2


def _body(t_ref, c1_ref, c2_ref, x_ref, n_ref, o_ref):
    i = pl.program_id(0)
    for r in range(_ROWS):
        ti = t_ref[i * _ROWS + r]
        a = c1_ref[ti]
        b = c2_ref[ti]
        o_ref[r, :] = a * x_ref[r, :] + b * n_ref[r, :]


@jax.jit
def kernel(x, noise, t):
    B = x.shape[0]
    c1, c2 = _schedule_tables()
    x2 = x.reshape(B, _COLS)
    n2 = noise.reshape(B, _COLS)
    grid = (B // _ROWS,)
    out = pl.pallas_call(
        _body,
        grid_spec=pltpu.PrefetchScalarGridSpec(
            num_scalar_prefetch=3,
            grid=grid,
            in_specs=[
                pl.BlockSpec((_ROWS, _COLS), lambda i, *_: (i, 0)),
                pl.BlockSpec((_ROWS, _COLS), lambda i, *_: (i, 0)),
            ],
            out_specs=pl.BlockSpec((_ROWS, _COLS), lambda i, *_: (i, 0)),
        ),
        out_shape=jax.ShapeDtypeStruct((B, _COLS), jnp.float32),
    )(t.astype(jnp.int32), c1, c2, x2, n2)
    return out.reshape(x.shape)
